# Initial kernel scaffold; baseline (speedup 1.0000x reference)
import jax
import jax.numpy as jnp
from jax.experimental import pallas as pl


def kernel(x, Wq, Wk, Wv, Wp, bp, pos_embed):
    raise NotImplementedError("write your pallas kernel here")


# speedup vs baseline: 1.7222x; 1.7222x over previous
import functools

import jax
import jax.numpy as jnp
from jax import lax
from jax.experimental import pallas as pl
from jax.experimental.pallas import tpu as pltpu
from jax.experimental.pallas import tpu_sc as plsc

B, N, C = 64, 243, 512
HEADS = 8
HD = C // HEADS
CLUSTER = 81
KNN = 7
NPAD = 256
KPAD = 96
RSEL = 128
SQRT_C = float(C) ** 0.5
NEG_INF = float("-inf")


def _rowize(v, iota_r, iota_c):
    sel = iota_r == iota_c
    return jnp.sum(jnp.where(sel, jnp.broadcast_to(v, (NPAD, NPAD)), 0.0),
                   axis=0, keepdims=True)


def _score_body(x_ref, idx_ref):
    b = pl.program_id(0)
    xb = x_ref[0]
    iota_r = lax.broadcasted_iota(jnp.int32, (NPAD, NPAD), 0)
    iota_c = lax.broadcasted_iota(jnp.int32, (NPAD, NPAD), 1)

    n2 = jnp.sum(xb * xb, axis=1, keepdims=True)
    n2r = _rowize(n2, iota_r, iota_c)
    g = lax.dot_general(xb, xb, (((1,), (1,)), ((), ())),
                        preferred_element_type=jnp.float32)
    d2 = n2 + n2r - 2.0 * g
    dist = jnp.sqrt(jnp.maximum(d2, 0.0)) / SQRT_C

    colvalid = iota_c < N
    rowvalid = iota_r < N

    inf = jnp.float32(jnp.inf)
    work = jnp.where(colvalid, dist, inf)
    acc = jnp.zeros((NPAD, 1), jnp.float32)
    for _ in range(KNN):
        m = jnp.min(work, axis=1, keepdims=True)
        first = jnp.min(jnp.where(work == m, iota_c, NPAD), axis=1, keepdims=True)
        acc = acc + m * m
        work = jnp.where(iota_c == first, inf, work)
    density = jnp.exp(-(acc / KNN))
    density = density + lax.broadcasted_iota(
        jnp.int32, (NPAD, 1), 0).astype(jnp.float32) * jnp.float32(1e-6)
    densr = _rowize(density, iota_r, iota_c)

    distmax = jnp.max(jnp.where(rowvalid & colvalid, dist, -inf))
    mask = (densr > density) & colvalid
    parent = jnp.min(jnp.where(mask, dist, distmax), axis=1, keepdims=True)

    score = parent * density
    score = jnp.where(lax.broadcasted_iota(jnp.int32, (NPAD, 1), 0) < N,
                      score, -inf)
    scorer = _rowize(score, iota_r, iota_c)

    beat = (scorer > score) | ((scorer == score) & (iota_c < iota_r))
    rank = jnp.sum(beat.astype(jnp.int32), axis=1, keepdims=True)

    iota_rr = lax.broadcasted_iota(jnp.int32, (NPAD, RSEL), 1)
    ival = lax.broadcasted_iota(jnp.int32, (NPAD, RSEL), 0) + b * N
    out = jnp.sum(jnp.where(rank == iota_rr, ival, 0), axis=0, keepdims=True)
    idx_ref[...] = out.reshape(1, 1, RSEL)


def _score_call(x_pad):
    return pl.pallas_call(
        _score_body,
        grid=(B,),
        in_specs=[pl.BlockSpec((1, NPAD, C), lambda b: (b, 0, 0))],
        out_specs=pl.BlockSpec((1, 1, RSEL), lambda b: (b, 0, 0)),
        out_shape=jax.ShapeDtypeStruct((B, 1, RSEL), jnp.int32),
    )(x_pad)


_NC = 2
_NS = 16
_NW = _NC * _NS
_ROWS_TOTAL = B * KPAD
_PER_CH = KPAD
_CH_PER_W = _ROWS_TOTAL // (_NW * _PER_CH)
_PER_W = _CH_PER_W * _PER_CH


def _gather_body(xflat_hbm, idx_hbm, out_hbm, idx_v, rows_v, sem):
    wid = lax.axis_index("s") * _NC + lax.axis_index("c")
    base = wid * _CH_PER_W
    pltpu.sync_copy(idx_hbm.at[pl.ds(base, _CH_PER_W)], idx_v)
    cps = [
        pltpu.async_copy(xflat_hbm.at[idx_v.at[j]],
                         rows_v.at[pl.ds(j * _PER_CH, _PER_CH)], sem)
        for j in range(_CH_PER_W)
    ]
    for cp in cps:
        cp.wait()
    pltpu.sync_copy(rows_v, out_hbm.at[pl.ds(base * _PER_CH, _PER_W)])


@functools.cache
def _gather_call():
    return pl.kernel(
        _gather_body,
        out_type=jax.ShapeDtypeStruct((_ROWS_TOTAL, C), jnp.float32),
        mesh=plsc.VectorSubcoreMesh(core_axis_name="c", subcore_axis_name="s"),
        scratch_types=[
            pltpu.VMEM((_CH_PER_W, _PER_CH), jnp.int32),
            pltpu.VMEM((_PER_W, C), jnp.float32),
            pltpu.SemaphoreType.DMA,
        ],
    )


def _attn_body(x_ref, cen_ref, pos_ref, wq_ref, wk_ref, wv_ref, wp_ref,
               bp_ref, o_ref):
    xb = x_ref[0]
    cen = cen_ref[0] + pos_ref[0]
    q = jnp.dot(xb, wq_ref[...], preferred_element_type=jnp.float32)
    k = jnp.dot(cen, wk_ref[...], preferred_element_type=jnp.float32)
    v = jnp.dot(cen, wv_ref[...], preferred_element_type=jnp.float32)
    scale = jnp.float32(HD ** -0.5)
    kmask = lax.broadcasted_iota(jnp.int32, (NPAD, KPAD), 1) < CLUSTER
    outs = []
    for h in range(HEADS):
        qh = q[:, h * HD:(h + 1) * HD]
        kh = k[:, h * HD:(h + 1) * HD]
        vh = v[:, h * HD:(h + 1) * HD]
        s = lax.dot_general(qh, kh, (((1,), (1,)), ((), ())),
                            preferred_element_type=jnp.float32) * scale
        s = jnp.where(kmask, s, NEG_INF)
        m = jnp.max(s, axis=1, keepdims=True)
        e = jnp.exp(s - m)
        p = e / jnp.sum(e, axis=1, keepdims=True)
        outs.append(jnp.dot(p, vh, preferred_element_type=jnp.float32))
    o = jnp.concatenate(outs, axis=1)
    res = jnp.dot(o, wp_ref[...], preferred_element_type=jnp.float32)
    o_ref[0] = res + bp_ref[...] + xb


def _attn_call(x_pad, centers, pos_pad, Wq, Wk, Wv, Wp, bp2):
    wspec = pl.BlockSpec((C, C), lambda b: (0, 0))
    return pl.pallas_call(
        _attn_body,
        grid=(B,),
        in_specs=[
            pl.BlockSpec((1, NPAD, C), lambda b: (b, 0, 0)),
            pl.BlockSpec((1, KPAD, C), lambda b: (b, 0, 0)),
            pl.BlockSpec((1, KPAD, C), lambda b: (0, 0, 0)),
            wspec, wspec, wspec, wspec,
            pl.BlockSpec((1, C), lambda b: (0, 0)),
        ],
        out_specs=pl.BlockSpec((1, NPAD, C), lambda b: (b, 0, 0)),
        out_shape=jax.ShapeDtypeStruct((B, NPAD, C), jnp.float32),
    )(x_pad, centers, pos_pad, Wq, Wk, Wv, Wp, bp2)


def kernel(x, Wq, Wk, Wv, Wp, bp, pos_embed):
    x_pad = jnp.pad(x, ((0, 0), (0, NPAD - N), (0, 0)))
    idx3 = _score_call(x_pad)
    idx2 = idx3[:, 0, :KPAD]
    gathered = _gather_call()(x.reshape(B * N, C), idx2)
    centers = gathered.reshape(B, KPAD, C)
    pos_pad = jnp.pad(pos_embed, ((0, 0), (0, KPAD - CLUSTER), (0, 0)))
    out = _attn_call(x_pad, centers, pos_pad, Wq, Wk, Wv, Wp,
                     bp.reshape(1, C))
    return out[:, :N, :]


# speedup vs baseline: 1.7382x; 1.0093x over previous
import functools

import jax
import jax.numpy as jnp
from jax import lax
from jax.experimental import pallas as pl
from jax.experimental.pallas import tpu as pltpu
from jax.experimental.pallas import tpu_sc as plsc

B, N, C = 64, 243, 512
HEADS = 8
HD = C // HEADS
CLUSTER = 81
KNN = 7
KPAD = 96
RSEL = 128
SQRT_C = float(C) ** 0.5
NEG_INF = float("-inf")


def _rowize(v, iota_r, iota_c):
    sel = iota_r == iota_c
    return jnp.sum(jnp.where(sel, jnp.broadcast_to(v, (N, N)), 0.0),
                   axis=0, keepdims=True)


def _score_body(x_ref, idx_ref):
    xb = x_ref[0]
    iota_r = lax.broadcasted_iota(jnp.int32, (N, N), 0)
    iota_c = lax.broadcasted_iota(jnp.int32, (N, N), 1)

    n2 = jnp.sum(xb * xb, axis=1, keepdims=True)
    n2r = _rowize(n2, iota_r, iota_c)
    g = lax.dot_general(xb, xb, (((1,), (1,)), ((), ())),
                        preferred_element_type=jnp.float32)
    d2 = n2 + n2r - 2.0 * g
    dist = jnp.sqrt(jnp.maximum(d2, 0.0)) / SQRT_C

    inf = jnp.float32(jnp.inf)
    work = dist
    acc = jnp.zeros((N, 1), jnp.float32)
    for _ in range(KNN):
        m = jnp.min(work, axis=1, keepdims=True)
        first = jnp.min(jnp.where(work == m, iota_c, N), axis=1, keepdims=True)
        acc = acc + m * m
        work = jnp.where(iota_c == first, inf, work)
    density = jnp.exp(-(acc / KNN))
    density = density + lax.broadcasted_iota(
        jnp.int32, (N, 1), 0).astype(jnp.float32) * jnp.float32(1e-6)
    densr = _rowize(density, iota_r, iota_c)

    distmax = jnp.max(dist)
    parent = jnp.min(jnp.where(densr > density, dist, distmax),
                     axis=1, keepdims=True)

    score = parent * density
    scorer = _rowize(score, iota_r, iota_c)

    beat = (scorer > score) | ((scorer == score) & (iota_c < iota_r))
    rank = jnp.sum(beat.astype(jnp.int32), axis=1, keepdims=True)

    iota_rr = lax.broadcasted_iota(jnp.int32, (N, RSEL), 1)
    ival = lax.broadcasted_iota(jnp.int32, (N, RSEL), 0)
    out = jnp.sum(jnp.where(rank == iota_rr, ival, 0), axis=0, keepdims=True)
    idx_ref[...] = out.reshape(1, 1, RSEL)


def _score_call(x):
    return pl.pallas_call(
        _score_body,
        grid=(B,),
        in_specs=[pl.BlockSpec((1, N, C), lambda b: (b, 0, 0))],
        out_specs=pl.BlockSpec((1, 1, RSEL), lambda b: (b, 0, 0)),
        out_shape=jax.ShapeDtypeStruct((B, 1, RSEL), jnp.int32),
    )(x)


_NC = 2
_NS = 16
_NW = _NC * _NS
_B_PER_W = B // _NW
_PER_W = _B_PER_W * KPAD


def _gather_body(x_hbm, idx_hbm, out_hbm, idx_v, rows_v, sem):
    wid = lax.axis_index("s") * _NC + lax.axis_index("c")
    b0 = wid * _B_PER_W
    pltpu.sync_copy(idx_hbm.at[pl.ds(b0, _B_PER_W)], idx_v)
    cps = [
        pltpu.async_copy(x_hbm.at[b0 + j].at[idx_v.at[j]],
                         rows_v.at[pl.ds(j * KPAD, KPAD)], sem)
        for j in range(_B_PER_W)
    ]
    for cp in cps:
        cp.wait()
    pltpu.sync_copy(rows_v, out_hbm.at[pl.ds(b0 * KPAD, _PER_W)])


@functools.cache
def _gather_call():
    return pl.kernel(
        _gather_body,
        out_type=jax.ShapeDtypeStruct((B * KPAD, C), jnp.float32),
        mesh=plsc.VectorSubcoreMesh(core_axis_name="c", subcore_axis_name="s"),
        scratch_types=[
            pltpu.VMEM((_B_PER_W, KPAD), jnp.int32),
            pltpu.VMEM((_PER_W, C), jnp.float32),
            pltpu.SemaphoreType.DMA,
        ],
    )


def _attn_body(x_ref, cen_ref, pos_ref, wq_ref, wk_ref, wv_ref, wp_ref,
               bp_ref, o_ref):
    xb = x_ref[0]
    cen = cen_ref[...] + pos_ref[0]
    q = jnp.dot(xb, wq_ref[...], preferred_element_type=jnp.float32)
    k = jnp.dot(cen, wk_ref[...], preferred_element_type=jnp.float32)
    v = jnp.dot(cen, wv_ref[...], preferred_element_type=jnp.float32)
    scale = jnp.float32(HD ** -0.5)
    kmask = lax.broadcasted_iota(jnp.int32, (N, KPAD), 1) < CLUSTER
    outs = []
    for h in range(HEADS):
        qh = q[:, h * HD:(h + 1) * HD]
        kh = k[:, h * HD:(h + 1) * HD]
        vh = v[:, h * HD:(h + 1) * HD]
        s = lax.dot_general(qh, kh, (((1,), (1,)), ((), ())),
                            preferred_element_type=jnp.float32) * scale
        s = jnp.where(kmask, s, NEG_INF)
        m = jnp.max(s, axis=1, keepdims=True)
        e = jnp.exp(s - m)
        p = e / jnp.sum(e, axis=1, keepdims=True)
        outs.append(jnp.dot(p, vh, preferred_element_type=jnp.float32))
    o = jnp.concatenate(outs, axis=1)
    res = jnp.dot(o, wp_ref[...], preferred_element_type=jnp.float32)
    o_ref[0] = res + bp_ref[...] + xb


def _attn_call(x, centers_flat, pos_pad, Wq, Wk, Wv, Wp, bp2):
    wspec = pl.BlockSpec((C, C), lambda b: (0, 0))
    return pl.pallas_call(
        _attn_body,
        grid=(B,),
        in_specs=[
            pl.BlockSpec((1, N, C), lambda b: (b, 0, 0)),
            pl.BlockSpec((KPAD, C), lambda b: (b, 0)),
            pl.BlockSpec((1, KPAD, C), lambda b: (0, 0, 0)),
            wspec, wspec, wspec, wspec,
            pl.BlockSpec((1, C), lambda b: (0, 0)),
        ],
        out_specs=pl.BlockSpec((1, N, C), lambda b: (b, 0, 0)),
        out_shape=jax.ShapeDtypeStruct((B, N, C), jnp.float32),
    )(x, centers_flat, pos_pad, Wq, Wk, Wv, Wp, bp2)


def kernel(x, Wq, Wk, Wv, Wp, bp, pos_embed):
    idx3 = _score_call(x)
    idx2 = idx3[:, 0, :KPAD]
    centers_flat = _gather_call()(x, idx2)
    pos_pad = jnp.pad(pos_embed, ((0, 0), (0, KPAD - CLUSTER), (0, 0)))
    return _attn_call(x, centers_flat, pos_pad, Wq, Wk, Wv, Wp,
                      bp.reshape(1, C))


# speedup vs baseline: 1.9157x; 1.1021x over previous
import functools

import jax
import jax.numpy as jnp
from jax import lax
from jax.experimental import pallas as pl
from jax.experimental.pallas import tpu as pltpu
from jax.experimental.pallas import tpu_sc as plsc

B, N, C = 64, 243, 512
HEADS = 8
HD = C // HEADS
CLUSTER = 81
KNN = 7
NPAD = 256
KPAD = 96
RSEL = 128
SQRT_C = float(C) ** 0.5
NEG_INF = float("-inf")
_KEYMASK = -256
_KEYINF = 0x7F800000


def _rowize(v, iota_r, iota_c):
    sel = iota_r == iota_c
    return jnp.sum(jnp.where(sel, jnp.broadcast_to(v, (NPAD, NPAD)), 0.0),
                   axis=0, keepdims=True)


def _score_body(x_ref, idx_ref):
    xb = x_ref[0]
    iota_r = lax.broadcasted_iota(jnp.int32, (NPAD, NPAD), 0)
    iota_c = lax.broadcasted_iota(jnp.int32, (NPAD, NPAD), 1)
    colvalid = iota_c < N
    inf = jnp.float32(jnp.inf)

    n2 = jnp.sum(xb * xb, axis=1, keepdims=True)
    n2r = _rowize(n2, iota_r, iota_c)
    g = lax.dot_general(xb, xb, (((1,), (1,)), ((), ())),
                        preferred_element_type=jnp.float32)
    d2 = n2 + n2r - 2.0 * g
    d2c = jnp.maximum(d2, 0.0)

    keymask = jnp.int32(_KEYMASK)
    keyinf = jnp.int32(_KEYINF)
    bits = lax.bitcast_convert_type(d2c, jnp.int32)
    key = jnp.where(colvalid, (bits & keymask) | iota_c, keyinf)
    acc = jnp.zeros((NPAD, 1), jnp.float32)
    for _ in range(KNN):
        kmin = jnp.min(key, axis=1, keepdims=True)
        vnear = lax.bitcast_convert_type(kmin & keymask, jnp.float32)
        dnear = jnp.sqrt(vnear) / SQRT_C
        acc = acc + dnear * dnear
        key = jnp.where(key == kmin, keyinf, key)
    density = jnp.exp(-(acc / KNN))
    density = density + lax.broadcasted_iota(
        jnp.int32, (NPAD, 1), 0).astype(jnp.float32) * jnp.float32(1e-6)
    densr = _rowize(density, iota_r, iota_c)

    d2max = jnp.max(jnp.where((iota_r < N) & colvalid, d2c, -inf))
    mask = (densr > density) & colvalid
    parent_d2 = jnp.min(jnp.where(mask, d2c, d2max), axis=1, keepdims=True)
    parent = jnp.sqrt(parent_d2) / SQRT_C

    score = parent * density
    score = jnp.where(lax.broadcasted_iota(jnp.int32, (NPAD, 1), 0) < N,
                      score, -inf)
    scorer = _rowize(score, iota_r, iota_c)

    beat = (scorer > score) | ((scorer == score) & (iota_c < iota_r))
    rank = jnp.sum(beat.astype(jnp.int32), axis=1, keepdims=True)

    iota_rr = lax.broadcasted_iota(jnp.int32, (NPAD, RSEL), 1)
    ival = lax.broadcasted_iota(jnp.int32, (NPAD, RSEL), 0)
    out = jnp.sum(jnp.where(rank == iota_rr, ival, 0), axis=0, keepdims=True)
    idx_ref[...] = out.reshape(1, 1, RSEL)


def _score_call(x_pad):
    return pl.pallas_call(
        _score_body,
        grid=(B,),
        in_specs=[pl.BlockSpec((1, NPAD, C), lambda b: (b, 0, 0))],
        out_specs=pl.BlockSpec((1, 1, RSEL), lambda b: (b, 0, 0)),
        out_shape=jax.ShapeDtypeStruct((B, 1, RSEL), jnp.int32),
    )(x_pad)


_NC = 2
_NS = 16
_NW = _NC * _NS
_B_PER_W = B // _NW
_PER_W = _B_PER_W * KPAD


def _gather_body(x_hbm, idx_hbm, out_hbm, idx_v, rows_v, sem):
    wid = lax.axis_index("s") * _NC + lax.axis_index("c")
    b0 = wid * _B_PER_W
    pltpu.sync_copy(idx_hbm.at[pl.ds(b0, _B_PER_W)], idx_v)
    cps = [
        pltpu.async_copy(x_hbm.at[b0 + j].at[idx_v.at[j]],
                         rows_v.at[pl.ds(j * KPAD, KPAD)], sem)
        for j in range(_B_PER_W)
    ]
    for cp in cps:
        cp.wait()
    pltpu.sync_copy(rows_v, out_hbm.at[pl.ds(b0 * KPAD, _PER_W)])


@functools.cache
def _gather_call():
    return pl.kernel(
        _gather_body,
        out_type=jax.ShapeDtypeStruct((B * KPAD, C), jnp.float32),
        mesh=plsc.VectorSubcoreMesh(core_axis_name="c", subcore_axis_name="s"),
        scratch_types=[
            pltpu.VMEM((_B_PER_W, KPAD), jnp.int32),
            pltpu.VMEM((_PER_W, C), jnp.float32),
            pltpu.SemaphoreType.DMA,
        ],
    )


def _attn_body(x_ref, cen_ref, pos_ref, wq_ref, wk_ref, wv_ref, wp_ref,
               bp_ref, o_ref):
    xb = x_ref[0]
    cen = cen_ref[...] + pos_ref[0]
    q = jnp.dot(xb, wq_ref[...], preferred_element_type=jnp.float32)
    k = jnp.dot(cen, wk_ref[...], preferred_element_type=jnp.float32)
    v = jnp.dot(cen, wv_ref[...], preferred_element_type=jnp.float32)
    scale = jnp.float32(HD ** -0.5)
    kmask = lax.broadcasted_iota(jnp.int32, (NPAD, KPAD), 1) < CLUSTER
    outs = []
    for h in range(HEADS):
        qh = q[:, h * HD:(h + 1) * HD]
        kh = k[:, h * HD:(h + 1) * HD]
        vh = v[:, h * HD:(h + 1) * HD]
        s = lax.dot_general(qh, kh, (((1,), (1,)), ((), ())),
                            preferred_element_type=jnp.float32) * scale
        s = jnp.where(kmask, s, NEG_INF)
        m = jnp.max(s, axis=1, keepdims=True)
        e = jnp.exp(s - m)
        p = e / jnp.sum(e, axis=1, keepdims=True)
        outs.append(jnp.dot(p, vh, preferred_element_type=jnp.float32))
    o = jnp.concatenate(outs, axis=1)
    res = jnp.dot(o, wp_ref[...], preferred_element_type=jnp.float32)
    o_ref[0] = res + bp_ref[...] + xb


def _attn_call(x_pad, centers_flat, pos_pad, Wq, Wk, Wv, Wp, bp2):
    wspec = pl.BlockSpec((C, C), lambda b: (0, 0))
    return pl.pallas_call(
        _attn_body,
        grid=(B,),
        in_specs=[
            pl.BlockSpec((1, NPAD, C), lambda b: (b, 0, 0)),
            pl.BlockSpec((KPAD, C), lambda b: (b, 0)),
            pl.BlockSpec((1, KPAD, C), lambda b: (0, 0, 0)),
            wspec, wspec, wspec, wspec,
            pl.BlockSpec((1, C), lambda b: (0, 0)),
        ],
        out_specs=pl.BlockSpec((1, NPAD, C), lambda b: (b, 0, 0)),
        out_shape=jax.ShapeDtypeStruct((B, NPAD, C), jnp.float32),
    )(x_pad, centers_flat, pos_pad, Wq, Wk, Wv, Wp, bp2)


def kernel(x, Wq, Wk, Wv, Wp, bp, pos_embed):
    x_pad = jnp.pad(x, ((0, 0), (0, NPAD - N), (0, 0)))
    idx3 = _score_call(x_pad)
    idx2 = idx3[:, 0, :KPAD]
    centers_flat = _gather_call()(x, idx2)
    pos_pad = jnp.pad(pos_embed, ((0, 0), (0, KPAD - CLUSTER), (0, 0)))
    out = _attn_call(x_pad, centers_flat, pos_pad, Wq, Wk, Wv, Wp,
                     bp.reshape(1, C))
    return out[:, :N, :]


# speedup vs baseline: 2.0496x; 1.0699x over previous
import functools

import jax
import jax.numpy as jnp
from jax import lax
from jax.experimental import pallas as pl
from jax.experimental.pallas import tpu as pltpu
from jax.experimental.pallas import tpu_sc as plsc

B, N, C = 64, 243, 512
HEADS = 8
HD = C // HEADS
CLUSTER = 81
KNN = 7
NPAD = 256
KPAD = 96
RSEL = 128
SQRT_C = float(C) ** 0.5
NEG_INF = float("-inf")
_KEYMASK = -256
_KEYINF = 0x7F800000


def _rowize(v, iota_r, iota_c):
    sel = iota_r == iota_c
    return jnp.sum(jnp.where(sel, jnp.broadcast_to(v, (NPAD, NPAD)), 0.0),
                   axis=0, keepdims=True)


def _score_body(x_ref, idx_ref):
    xb = x_ref[0]
    iota_r = lax.broadcasted_iota(jnp.int32, (NPAD, NPAD), 0)
    iota_c = lax.broadcasted_iota(jnp.int32, (NPAD, NPAD), 1)
    colvalid = iota_c < N
    inf = jnp.float32(jnp.inf)

    n2 = jnp.sum(xb * xb, axis=1, keepdims=True)
    n2r = _rowize(n2, iota_r, iota_c)
    g = lax.dot_general(xb, xb, (((1,), (1,)), ((), ())),
                        preferred_element_type=jnp.float32)
    d2 = n2 + n2r - 2.0 * g
    d2c = jnp.maximum(d2, 0.0)

    keymask = jnp.int32(_KEYMASK)
    keyinf = jnp.int32(_KEYINF)
    bits = lax.bitcast_convert_type(d2c, jnp.int32)
    key = jnp.where(colvalid, (bits & keymask) | iota_c, keyinf)
    acc = jnp.zeros((NPAD, 1), jnp.float32)
    for _ in range(KNN):
        kmin = jnp.min(key, axis=1, keepdims=True)
        vnear = lax.bitcast_convert_type(kmin & keymask, jnp.float32)
        dnear = jnp.sqrt(vnear) / SQRT_C
        acc = acc + dnear * dnear
        key = jnp.where(key == kmin, keyinf, key)
    density = jnp.exp(-(acc / KNN))
    density = density + lax.broadcasted_iota(
        jnp.int32, (NPAD, 1), 0).astype(jnp.float32) * jnp.float32(1e-6)
    densr = _rowize(density, iota_r, iota_c)

    d2max = jnp.max(jnp.where((iota_r < N) & colvalid, d2c, -inf))
    mask = (densr > density) & colvalid
    parent_d2 = jnp.min(jnp.where(mask, d2c, d2max), axis=1, keepdims=True)
    parent = jnp.sqrt(parent_d2) / SQRT_C

    score = parent * density
    score = jnp.where(lax.broadcasted_iota(jnp.int32, (NPAD, 1), 0) < N,
                      score, -inf)
    scorer = _rowize(score, iota_r, iota_c)

    beat = (scorer > score) | ((scorer == score) & (iota_c < iota_r))
    rank = jnp.sum(beat.astype(jnp.int32), axis=1, keepdims=True)

    iota_rr = lax.broadcasted_iota(jnp.int32, (NPAD, RSEL), 1)
    ival = lax.broadcasted_iota(jnp.int32, (NPAD, RSEL), 0)
    out = jnp.sum(jnp.where(rank == iota_rr, ival, 0), axis=0, keepdims=True)
    idx_ref[...] = out.reshape(1, 1, RSEL)


def _score_call(x_pad):
    return pl.pallas_call(
        _score_body,
        grid=(B,),
        in_specs=[pl.BlockSpec((1, NPAD, C), lambda b: (b, 0, 0))],
        out_specs=pl.BlockSpec((1, 1, RSEL), lambda b: (b, 0, 0)),
        out_shape=jax.ShapeDtypeStruct((B, 1, RSEL), jnp.int32),
    )(x_pad)


_NC = 2
_NS = 16
_NW = _NC * _NS
_B_PER_W = B // _NW
_PER_W = _B_PER_W * KPAD


def _gather_body(x_hbm, idx_hbm, out_hbm, idx_v, rows_v, sem):
    wid = lax.axis_index("s") * _NC + lax.axis_index("c")
    b0 = wid * _B_PER_W
    pltpu.sync_copy(idx_hbm.at[pl.ds(b0, _B_PER_W)], idx_v)
    cps = [
        pltpu.async_copy(x_hbm.at[b0 + j].at[idx_v.at[j]],
                         rows_v.at[pl.ds(j * KPAD, KPAD)], sem)
        for j in range(_B_PER_W)
    ]
    for cp in cps:
        cp.wait()
    pltpu.sync_copy(rows_v, out_hbm.at[pl.ds(b0 * KPAD, _PER_W)])


@functools.cache
def _gather_call():
    return pl.kernel(
        _gather_body,
        out_type=jax.ShapeDtypeStruct((B * KPAD, C), jnp.float32),
        mesh=plsc.VectorSubcoreMesh(core_axis_name="c", subcore_axis_name="s"),
        scratch_types=[
            pltpu.VMEM((_B_PER_W, KPAD), jnp.int32),
            pltpu.VMEM((_PER_W, C), jnp.float32),
            pltpu.SemaphoreType.DMA,
        ],
    )


def _attn_body(x_ref, cen_ref, pos_ref, wq_ref, wk_ref, wv_ref, wp_ref,
               bp_ref, o_ref):
    xb = x_ref[0]
    cen = cen_ref[...] + pos_ref[0]
    q = jnp.dot(xb, wq_ref[...], preferred_element_type=jnp.float32)
    k = jnp.dot(cen, wk_ref[...], preferred_element_type=jnp.float32)
    v = jnp.dot(cen, wv_ref[...], preferred_element_type=jnp.float32)
    scale = jnp.float32(HD ** -0.5)
    kmask = lax.broadcasted_iota(jnp.int32, (NPAD, KPAD), 1) < CLUSTER
    outs = []
    for h in range(HEADS):
        qh = q[:, h * HD:(h + 1) * HD]
        kh = k[:, h * HD:(h + 1) * HD]
        vh = v[:, h * HD:(h + 1) * HD]
        s = lax.dot_general(qh, kh, (((1,), (1,)), ((), ())),
                            preferred_element_type=jnp.float32) * scale
        s = jnp.where(kmask, s, NEG_INF)
        m = jnp.max(s, axis=1, keepdims=True)
        e = jnp.exp(s - m)
        p = e / jnp.sum(e, axis=1, keepdims=True)
        outs.append(jnp.dot(p, vh, preferred_element_type=jnp.float32))
    o = jnp.concatenate(outs, axis=1)
    res = jnp.dot(o, wp_ref[...], preferred_element_type=jnp.float32)
    o_ref[0] = (res + bp_ref[...] + xb)[:N]


def _attn_call(x_pad, centers_flat, pos_pad, Wq, Wk, Wv, Wp, bp2):
    wspec = pl.BlockSpec((C, C), lambda b: (0, 0))
    return pl.pallas_call(
        _attn_body,
        grid=(B,),
        in_specs=[
            pl.BlockSpec((1, NPAD, C), lambda b: (b, 0, 0)),
            pl.BlockSpec((KPAD, C), lambda b: (b, 0)),
            pl.BlockSpec((1, KPAD, C), lambda b: (0, 0, 0)),
            wspec, wspec, wspec, wspec,
            pl.BlockSpec((1, C), lambda b: (0, 0)),
        ],
        out_specs=pl.BlockSpec((1, N, C), lambda b: (b, 0, 0)),
        out_shape=jax.ShapeDtypeStruct((B, N, C), jnp.float32),
    )(x_pad, centers_flat, pos_pad, Wq, Wk, Wv, Wp, bp2)


def kernel(x, Wq, Wk, Wv, Wp, bp, pos_embed):
    x_pad = jnp.pad(x, ((0, 0), (0, NPAD - N), (0, 0)))
    idx3 = _score_call(x_pad)
    idx2 = idx3[:, 0, :KPAD]
    centers_flat = _gather_call()(x, idx2)
    pos_pad = jnp.pad(pos_embed, ((0, 0), (0, KPAD - CLUSTER), (0, 0)))
    return _attn_call(x_pad, centers_flat, pos_pad, Wq, Wk, Wv, Wp,
                      bp.reshape(1, C))


# speedup vs baseline: 2.0569x; 1.0036x over previous
import functools

import jax
import jax.numpy as jnp
from jax import lax
from jax.experimental import pallas as pl
from jax.experimental.pallas import tpu as pltpu
from jax.experimental.pallas import tpu_sc as plsc

B, N, C = 64, 243, 512
HEADS = 8
HD = C // HEADS
CLUSTER = 81
KNN = 7
NPAD = 256
KPAD = 96
RSEL = 128
SQRT_C = float(C) ** 0.5
NEG_INF = float("-inf")
_KEYMASK = -256
_KEYINF = 0x7F800000


def _rowize(v, iota_r, iota_c):
    sel = iota_r == iota_c
    return jnp.sum(jnp.where(sel, jnp.broadcast_to(v, (NPAD, NPAD)), 0.0),
                   axis=0, keepdims=True)


def _score_body(x_ref, idx_ref):
    xb = x_ref[0]
    iota_r = lax.broadcasted_iota(jnp.int32, (NPAD, NPAD), 0)
    iota_c = lax.broadcasted_iota(jnp.int32, (NPAD, NPAD), 1)
    colvalid = iota_c < N
    inf = jnp.float32(jnp.inf)

    n2 = jnp.sum(xb * xb, axis=1, keepdims=True)
    n2r = _rowize(n2, iota_r, iota_c)
    g = lax.dot_general(xb, xb, (((1,), (1,)), ((), ())),
                        preferred_element_type=jnp.float32)
    d2 = n2 + n2r - 2.0 * g
    d2c = jnp.maximum(d2, 0.0)

    keymask = jnp.int32(_KEYMASK)
    keyinf = jnp.int32(_KEYINF)
    bits = lax.bitcast_convert_type(d2c, jnp.int32)
    key = jnp.where(colvalid, (bits & keymask) | iota_c, keyinf)
    acc = jnp.zeros((NPAD, 1), jnp.float32)
    for _ in range(KNN):
        kmin = jnp.min(key, axis=1, keepdims=True)
        vnear = lax.bitcast_convert_type(kmin & keymask, jnp.float32)
        dnear = jnp.sqrt(vnear) / SQRT_C
        acc = acc + dnear * dnear
        key = jnp.where(key == kmin, keyinf, key)
    density = jnp.exp(-(acc / KNN))
    density = density + lax.broadcasted_iota(
        jnp.int32, (NPAD, 1), 0).astype(jnp.float32) * jnp.float32(1e-6)
    densr = _rowize(density, iota_r, iota_c)

    d2max = jnp.max(jnp.where((iota_r < N) & colvalid, d2c, -inf))
    mask = (densr > density) & colvalid
    parent_d2 = jnp.min(jnp.where(mask, d2c, d2max), axis=1, keepdims=True)
    parent = jnp.sqrt(parent_d2) / SQRT_C

    score = parent * density
    score = jnp.where(lax.broadcasted_iota(jnp.int32, (NPAD, 1), 0) < N,
                      score, -inf)
    scorer = _rowize(score, iota_r, iota_c)

    beat = (scorer > score) | ((scorer == score) & (iota_c < iota_r))
    rank = jnp.sum(beat.astype(jnp.int32), axis=1, keepdims=True)

    iota_rr = lax.broadcasted_iota(jnp.int32, (NPAD, RSEL), 1)
    ival = lax.broadcasted_iota(jnp.int32, (NPAD, RSEL), 0)
    out = jnp.sum(jnp.where(rank == iota_rr, ival, 0), axis=0, keepdims=True)
    idx_ref[...] = out.reshape(1, 1, RSEL)


def _score_call(x_pad):
    return pl.pallas_call(
        _score_body,
        grid=(B,),
        in_specs=[pl.BlockSpec((1, NPAD, C), lambda b: (b, 0, 0))],
        out_specs=pl.BlockSpec((1, 1, RSEL), lambda b: (b, 0, 0)),
        out_shape=jax.ShapeDtypeStruct((B, 1, RSEL), jnp.int32),
    )(x_pad)


_NC = 2
_NS = 16
_NW = _NC * _NS
_B_PER_W = B // _NW
_PER_W = _B_PER_W * KPAD


def _gather_body(x_hbm, idx_hbm, out_hbm, idx_v, rows_v, sem):
    wid = lax.axis_index("s") * _NC + lax.axis_index("c")
    b0 = wid * _B_PER_W
    pltpu.sync_copy(idx_hbm.at[pl.ds(b0, _B_PER_W)], idx_v)
    cps = [
        pltpu.async_copy(x_hbm.at[b0 + j].at[idx_v.at[j]],
                         rows_v.at[pl.ds(j * KPAD, KPAD)], sem)
        for j in range(_B_PER_W)
    ]
    for cp in cps:
        cp.wait()
    pltpu.sync_copy(rows_v, out_hbm.at[pl.ds(b0 * KPAD, _PER_W)])


@functools.cache
def _gather_call():
    return pl.kernel(
        _gather_body,
        out_type=jax.ShapeDtypeStruct((B * KPAD, C), jnp.float32),
        mesh=plsc.VectorSubcoreMesh(core_axis_name="c", subcore_axis_name="s"),
        scratch_types=[
            pltpu.VMEM((_B_PER_W, KPAD), jnp.int32),
            pltpu.VMEM((_PER_W, C), jnp.float32),
            pltpu.SemaphoreType.DMA,
        ],
    )


def _attn_body(x_ref, cen_ref, pos_ref, wq_ref, wk_ref, wv_ref, wp_ref,
               bp_ref, o_ref):
    xb = x_ref[0]
    cen = cen_ref[...] + pos_ref[0]
    xb16 = xb.astype(jnp.bfloat16)
    cen16 = cen.astype(jnp.bfloat16)
    q = jnp.dot(xb16, wq_ref[...].astype(jnp.bfloat16),
                preferred_element_type=jnp.float32)
    k = jnp.dot(cen16, wk_ref[...].astype(jnp.bfloat16),
                preferred_element_type=jnp.float32)
    v = jnp.dot(cen16, wv_ref[...].astype(jnp.bfloat16),
                preferred_element_type=jnp.float32).astype(jnp.bfloat16)
    scale = jnp.float32(HD ** -0.5)
    kmask = lax.broadcasted_iota(jnp.int32, (NPAD, KPAD), 1) < CLUSTER
    outs = []
    for h in range(HEADS):
        qh = q[:, h * HD:(h + 1) * HD].astype(jnp.bfloat16)
        kh = k[:, h * HD:(h + 1) * HD].astype(jnp.bfloat16)
        vh = v[:, h * HD:(h + 1) * HD]
        s = lax.dot_general(qh, kh, (((1,), (1,)), ((), ())),
                            preferred_element_type=jnp.float32) * scale
        s = jnp.where(kmask, s, NEG_INF)
        m = jnp.max(s, axis=1, keepdims=True)
        e = jnp.exp(s - m)
        p = (e / jnp.sum(e, axis=1, keepdims=True)).astype(jnp.bfloat16)
        outs.append(jnp.dot(p, vh, preferred_element_type=jnp.float32))
    o = jnp.concatenate(outs, axis=1).astype(jnp.bfloat16)
    res = jnp.dot(o, wp_ref[...].astype(jnp.bfloat16),
                  preferred_element_type=jnp.float32)
    o_ref[0] = (res + bp_ref[...] + xb)[:N]


def _attn_call(x_pad, centers_flat, pos_pad, Wq, Wk, Wv, Wp, bp2):
    wspec = pl.BlockSpec((C, C), lambda b: (0, 0))
    return pl.pallas_call(
        _attn_body,
        grid=(B,),
        in_specs=[
            pl.BlockSpec((1, NPAD, C), lambda b: (b, 0, 0)),
            pl.BlockSpec((KPAD, C), lambda b: (b, 0)),
            pl.BlockSpec((1, KPAD, C), lambda b: (0, 0, 0)),
            wspec, wspec, wspec, wspec,
            pl.BlockSpec((1, C), lambda b: (0, 0)),
        ],
        out_specs=pl.BlockSpec((1, N, C), lambda b: (b, 0, 0)),
        out_shape=jax.ShapeDtypeStruct((B, N, C), jnp.float32),
    )(x_pad, centers_flat, pos_pad, Wq, Wk, Wv, Wp, bp2)


def kernel(x, Wq, Wk, Wv, Wp, bp, pos_embed):
    x_pad = jnp.pad(x, ((0, 0), (0, NPAD - N), (0, 0)))
    idx3 = _score_call(x_pad)
    idx2 = idx3[:, 0, :KPAD]
    centers_flat = _gather_call()(x, idx2)
    pos_pad = jnp.pad(pos_embed, ((0, 0), (0, KPAD - CLUSTER), (0, 0)))
    return _attn_call(x_pad, centers_flat, pos_pad, Wq, Wk, Wv, Wp,
                      bp.reshape(1, C))


# speedup vs baseline: 2.5396x; 1.2347x over previous
import functools

import jax
import jax.numpy as jnp
from jax import lax
from jax.experimental import pallas as pl
from jax.experimental.pallas import tpu as pltpu
from jax.experimental.pallas import tpu_sc as plsc

B, N, C = 64, 243, 512
HEADS = 8
HD = C // HEADS
CLUSTER = 81
KNN = 7
NPAD = 256
KPAD = 96
RSEL = 128
SQRT_C = float(C) ** 0.5
NEG_INF = float("-inf")
_KEYMASK = -256
_KEYINF = 0x7F800000


def _rowize(v, iota_r, iota_c):
    sel = iota_r == iota_c
    return jnp.sum(jnp.where(sel, jnp.broadcast_to(v, (NPAD, NPAD)), 0.0),
                   axis=0, keepdims=True)


def _colize(v, iota_r, iota_c):
    sel = iota_r == iota_c
    return jnp.sum(jnp.where(sel, jnp.broadcast_to(v, (NPAD, NPAD)), 0.0),
                   axis=1, keepdims=True)


def _score_body(x_ref, idx_ref):
    xb = x_ref[0]
    iota_r = lax.broadcasted_iota(jnp.int32, (NPAD, NPAD), 0)
    iota_c = lax.broadcasted_iota(jnp.int32, (NPAD, NPAD), 1)
    rowvalid = iota_r < N
    inf = jnp.float32(jnp.inf)

    n2 = jnp.sum(xb * xb, axis=1, keepdims=True)
    n2r = _rowize(n2, iota_r, iota_c)
    g = lax.dot_general(xb, xb, (((1,), (1,)), ((), ())),
                        preferred_element_type=jnp.float32)
    d2 = n2 + n2r - 2.0 * g
    d2c = jnp.maximum(d2, 0.0)

    keymask = jnp.int32(_KEYMASK)
    keyinf = jnp.int32(_KEYINF)
    bits = lax.bitcast_convert_type(d2c, jnp.int32)
    key = jnp.where(rowvalid, (bits & keymask) | iota_r, keyinf)
    acc = jnp.zeros((1, NPAD), jnp.float32)
    for _ in range(KNN):
        kmin = jnp.min(key, axis=0, keepdims=True)
        vnear = lax.bitcast_convert_type(kmin & keymask, jnp.float32)
        dnear = jnp.sqrt(vnear) / SQRT_C
        acc = acc + dnear * dnear
        key = jnp.where(key == kmin, keyinf, key)
    densr = jnp.exp(-(acc / KNN))
    densr = densr + lax.broadcasted_iota(
        jnp.int32, (1, NPAD), 1).astype(jnp.float32) * jnp.float32(1e-6)
    densc = _colize(densr, iota_r, iota_c)

    colvalid = iota_c < N
    d2max = jnp.max(jnp.where(rowvalid & colvalid, d2c, -inf))
    mask = (densc > densr) & rowvalid
    parent_d2 = jnp.min(jnp.where(mask, d2c, d2max), axis=0, keepdims=True)
    parent = jnp.sqrt(parent_d2) / SQRT_C

    scorer = parent * densr
    scorer = jnp.where(lax.broadcasted_iota(jnp.int32, (1, NPAD), 1) < N,
                       scorer, -inf)
    score = _colize(scorer, iota_r, iota_c)

    beat = (scorer > score) | ((scorer == score) & (iota_c < iota_r))
    rank = jnp.sum(beat.astype(jnp.int32), axis=1, keepdims=True)

    iota_rr = lax.broadcasted_iota(jnp.int32, (NPAD, RSEL), 1)
    ival = lax.broadcasted_iota(jnp.int32, (NPAD, RSEL), 0)
    out = jnp.sum(jnp.where(rank == iota_rr, ival, 0), axis=0, keepdims=True)
    idx_ref[...] = out.reshape(1, 1, RSEL)


def _score_call(x_pad):
    return pl.pallas_call(
        _score_body,
        grid=(B,),
        in_specs=[pl.BlockSpec((1, NPAD, C), lambda b: (b, 0, 0))],
        out_specs=pl.BlockSpec((1, 1, RSEL), lambda b: (b, 0, 0)),
        out_shape=jax.ShapeDtypeStruct((B, 1, RSEL), jnp.int32),
    )(x_pad)


_NC = 2
_NS = 16
_NW = _NC * _NS
_B_PER_W = B // _NW
_PER_W = _B_PER_W * KPAD


def _gather_body(x_hbm, idx_hbm, out_hbm, idx_v, rows_v, sem):
    wid = lax.axis_index("s") * _NC + lax.axis_index("c")
    b0 = wid * _B_PER_W
    pltpu.sync_copy(idx_hbm.at[pl.ds(b0, _B_PER_W)], idx_v)
    cps = [
        pltpu.async_copy(x_hbm.at[b0 + j].at[idx_v.at[j]],
                         rows_v.at[pl.ds(j * KPAD, KPAD)], sem)
        for j in range(_B_PER_W)
    ]
    for cp in cps:
        cp.wait()
    pltpu.sync_copy(rows_v, out_hbm.at[pl.ds(b0 * KPAD, _PER_W)])


@functools.cache
def _gather_call():
    return pl.kernel(
        _gather_body,
        out_type=jax.ShapeDtypeStruct((B * KPAD, C), jnp.float32),
        mesh=plsc.VectorSubcoreMesh(core_axis_name="c", subcore_axis_name="s"),
        scratch_types=[
            pltpu.VMEM((_B_PER_W, KPAD), jnp.int32),
            pltpu.VMEM((_PER_W, C), jnp.float32),
            pltpu.SemaphoreType.DMA,
        ],
    )


def _attn_body(x_ref, cen_ref, pos_ref, wq_ref, wk_ref, wv_ref, wp_ref,
               bp_ref, o_ref):
    xb = x_ref[0]
    cen = cen_ref[...] + pos_ref[0]
    xb16 = xb.astype(jnp.bfloat16)
    cen16 = cen.astype(jnp.bfloat16)
    q = jnp.dot(xb16, wq_ref[...], preferred_element_type=jnp.float32)
    k = jnp.dot(cen16, wk_ref[...], preferred_element_type=jnp.float32)
    v = jnp.dot(cen16, wv_ref[...],
                preferred_element_type=jnp.float32).astype(jnp.bfloat16)
    scale = jnp.float32(HD ** -0.5)
    kmask = lax.broadcasted_iota(jnp.int32, (NPAD, KPAD), 1) < CLUSTER
    outs = []
    for h in range(HEADS):
        qh = q[:, h * HD:(h + 1) * HD].astype(jnp.bfloat16)
        kh = k[:, h * HD:(h + 1) * HD].astype(jnp.bfloat16)
        vh = v[:, h * HD:(h + 1) * HD]
        s = lax.dot_general(qh, kh, (((1,), (1,)), ((), ())),
                            preferred_element_type=jnp.float32) * scale
        e = jnp.where(kmask, jnp.exp(s), 0.0)
        r = 1.0 / jnp.sum(e, axis=1, keepdims=True)
        av = jnp.dot(e.astype(jnp.bfloat16), vh,
                     preferred_element_type=jnp.float32)
        outs.append(av * r)
    o = jnp.concatenate(outs, axis=1).astype(jnp.bfloat16)
    res = jnp.dot(o, wp_ref[...], preferred_element_type=jnp.float32)
    o_ref[0] = (res + bp_ref[...] + xb)[:N]


def _attn_call(x_pad, centers_flat, pos_pad, Wq, Wk, Wv, Wp, bp2):
    wspec = pl.BlockSpec((C, C), lambda b: (0, 0))
    return pl.pallas_call(
        _attn_body,
        grid=(B,),
        in_specs=[
            pl.BlockSpec((1, NPAD, C), lambda b: (b, 0, 0)),
            pl.BlockSpec((KPAD, C), lambda b: (b, 0)),
            pl.BlockSpec((1, KPAD, C), lambda b: (0, 0, 0)),
            wspec, wspec, wspec, wspec,
            pl.BlockSpec((1, C), lambda b: (0, 0)),
        ],
        out_specs=pl.BlockSpec((1, N, C), lambda b: (b, 0, 0)),
        out_shape=jax.ShapeDtypeStruct((B, N, C), jnp.float32),
    )(x_pad, centers_flat, pos_pad, Wq, Wk, Wv, Wp, bp2)


def kernel(x, Wq, Wk, Wv, Wp, bp, pos_embed):
    x_pad = jnp.pad(x, ((0, 0), (0, NPAD - N), (0, 0)))
    idx3 = _score_call(x_pad)
    idx2 = idx3[:, 0, :KPAD]
    centers_flat = _gather_call()(x, idx2)
    pos_pad = jnp.pad(pos_embed, ((0, 0), (0, KPAD - CLUSTER), (0, 0)))
    return _attn_call(x_pad, centers_flat, pos_pad,
                      Wq.astype(jnp.bfloat16), Wk.astype(jnp.bfloat16),
                      Wv.astype(jnp.bfloat16), Wp.astype(jnp.bfloat16),
                      bp.reshape(1, C))
